# baseline (speedup 1.0000x reference)
import jax
import jax.numpy as jnp
from jax.experimental import pallas as pl
from jax.experimental.pallas import tpu as pltpu

_C = 256
_HW = 4096
_BJ = 1024
_NK = _HW // _BJ
_AH = _HW // 2


def _nnfm_body(a_hbm, b_hbm, out_ref, a_buf, b_buf, a8_ref, sa, sb):
    a_cps = [
        pltpu.make_async_copy(
            a_hbm.at[:, pl.ds(h * _AH, _AH)],
            a_buf.at[:, pl.ds(h * _AH, _AH)], sa.at[h])
        for h in range(2)
    ]
    b_cps = [
        pltpu.make_async_copy(
            b_hbm.at[:, pl.ds(k * _BJ, _BJ)],
            b_buf.at[:, pl.ds(k * _BJ, _BJ)], sb.at[k])
        for k in range(_NK)
    ]
    a_cps[0].start()
    b_cps[0].start()
    a_cps[1].start()
    for k in range(1, _NK):
        b_cps[k].start()

    def _bn(k):
        bb = b_buf[:, k * _BJ:(k + 1) * _BJ]
        b_inv = jax.lax.rsqrt(jnp.sum(bb * bb, axis=0, keepdims=True) + 1e-16)
        return (bb * b_inv).astype(jnp.float8_e4m3fn)

    def _mm(h, bn):
        m = jax.lax.dot_general(
            a8_ref[:, h * _AH:(h + 1) * _AH], bn,
            (((0,), (0,)), ((), ())),
            preferred_element_type=jnp.float32)
        return jnp.max(m, axis=1, keepdims=True)

    rmax = [None, None]

    def _tile(h, bn):
        p = _mm(h, bn)
        rmax[h] = p if rmax[h] is None else jnp.maximum(rmax[h], p)

    a_cps[0].wait()
    a8_ref[:, :_AH] = a_buf[:, :_AH].astype(jnp.float8_e4m3fn)
    b_cps[0].wait()
    bn0 = _bn(0)
    _tile(0, bn0)
    a_cps[1].wait()
    a8_ref[:, _AH:] = a_buf[:, _AH:].astype(jnp.float8_e4m3fn)
    _tile(1, bn0)
    for k in range(1, _NK):
        b_cps[k].wait()
        bnk = _bn(k)
        _tile(0, bnk)
        _tile(1, bnk)

    t = 0.0
    for h in range(2):
        ah = a_buf[:, h * _AH:(h + 1) * _AH]
        a_inv = jax.lax.rsqrt(jnp.sum(ah * ah, axis=0, keepdims=True) + 1e-16)
        t = t + jax.lax.dot_general(
            a_inv, rmax[h], (((1,), (0,)), ((), ())),
            preferred_element_type=jnp.float32)
    out_ref[...] = 1.0 - t * (1.0 / _HW)


def kernel(outputs_feat, styles_feat):
    a = outputs_feat.reshape(_C, _HW)
    b = styles_feat.reshape(_C, _HW)
    out = pl.pallas_call(
        _nnfm_body,
        in_specs=[
            pl.BlockSpec(memory_space=pltpu.MemorySpace.HBM),
            pl.BlockSpec(memory_space=pltpu.MemorySpace.HBM),
        ],
        out_specs=pl.BlockSpec((1, 1), lambda: (0, 0)),
        out_shape=jax.ShapeDtypeStruct((1, 1), jnp.float32),
        scratch_shapes=[
            pltpu.VMEM((_C, _HW), jnp.float32),
            pltpu.VMEM((_C, _HW), jnp.float32),
            pltpu.VMEM((_C, _HW), jnp.float8_e4m3fn),
            pltpu.SemaphoreType.DMA((2,)),
            pltpu.SemaphoreType.DMA((_NK,)),
        ],
    )(a, b)
    return out[0, 0]
